# baseline (speedup 1.0000x reference)
import functools

import jax
import jax.numpy as jnp
import numpy as np
from jax import lax
from jax.experimental import pallas as pl
from jax.experimental.pallas import tpu as pltpu
from jax.experimental.pallas import tpu_sc as plsc

_N = 8388608
_NC = 2
_NS = 16
_NW = _NC * _NS
_L = 16

_EDGE10 = float(np.float32(1.0) + np.float32(1e-6))
_MOMENTUM = 0.5


def _make_sc_partials(n_total, chunk, interpret=False):
    per_w = n_total // _NW
    chunks = per_w // chunk
    vecs = chunk // _L
    assert per_w * _NW == n_total and chunks * chunk == per_w and vecs * _L == chunk

    assert chunks % 2 == 0

    def body(pred_hbm, target_hbm, out_hbm, pa, ta, pb, tb, acc_v, sem_a, sem_b):
        wid = lax.axis_index("s") * _NC + lax.axis_index("c")
        base = wid * per_w

        zeros = jnp.zeros((_L,), jnp.float32)
        ones = jnp.ones((_L,), jnp.float32)
        neg_big = jnp.full((_L,), -1e30, jnp.float32)

        def start(bp, bt, sem, c):
            s0 = base + c * chunk
            pltpu.async_copy(pred_hbm.at[pl.ds(s0, chunk)], bp, sem)
            pltpu.async_copy(target_hbm.at[pl.ds(s0, chunk)], bt, sem)

        def wait(bp, bt, sem, c):
            s0 = base + c * chunk
            pltpu.make_async_copy(pred_hbm.at[pl.ds(s0, chunk)], bp, sem).wait()
            pltpu.make_async_copy(target_hbm.at[pl.ds(s0, chunk)], bt, sem).wait()

        def compute(pbuf, tbuf, carry):
            M, S, B, A, Bgt, Agt, cnt = carry

            def pass1(i, c1):
                cm, B, A, Bgt, Agt, cnt = c1
                v = pbuf[pl.ds(i * _L, _L)]
                t = tbuf[pl.ds(i * _L, _L)]
                cm = jnp.maximum(cm, v)
                tp = t * v
                B = B + t
                A = A + tp
                gt = t > v
                Bgt = Bgt + jnp.where(gt, t, zeros)
                Agt = Agt + jnp.where(gt, tp, zeros)
                sg = 1.0 / (1.0 + jnp.exp(-v))
                g = jnp.abs(sg - t)
                cnt = cnt + jnp.where(g <= _EDGE10, ones, zeros)
                return (cm, B, A, Bgt, Agt, cnt)

            cm, B, A, Bgt, Agt, cnt = lax.fori_loop(
                0, vecs, pass1, (neg_big, B, A, Bgt, Agt, cnt), unroll=4)

            Mn = jnp.maximum(M, cm)
            S = S * jnp.exp(M - Mn)

            def pass2(i, s):
                v = pbuf[pl.ds(i * _L, _L)]
                return s + jnp.exp(v - Mn)

            S = lax.fori_loop(0, vecs, pass2, S, unroll=4)
            return (Mn, S, B, A, Bgt, Agt, cnt)

        start(pa, ta, sem_a, 0)

        def pair_body(i, carry):
            c0 = 2 * i
            start(pb, tb, sem_b, c0 + 1)
            wait(pa, ta, sem_a, c0)
            carry = compute(pa, ta, carry)

            @pl.when(c0 + 2 < chunks)
            def _():
                start(pa, ta, sem_a, c0 + 2)

            wait(pb, tb, sem_b, c0 + 1)
            carry = compute(pb, tb, carry)
            return carry

        init = (neg_big, zeros, zeros, zeros, zeros, zeros, zeros)
        M, S, B, A, Bgt, Agt, cnt = lax.fori_loop(0, chunks // 2, pair_body, init)

        acc_v[0] = M
        acc_v[1] = S
        acc_v[2] = B
        acc_v[3] = A
        acc_v[4] = Bgt
        acc_v[5] = Agt
        acc_v[6] = cnt
        acc_v[7] = zeros
        pltpu.sync_copy(acc_v, out_hbm.at[wid])

    return pl.kernel(
        body,
        out_type=jax.ShapeDtypeStruct((_NW, 8, _L), jnp.float32),
        mesh=plsc.VectorSubcoreMesh(
            core_axis_name="c", subcore_axis_name="s",
            num_cores=_NC, num_subcores=_NS),
        scratch_types=[
            pltpu.VMEM((chunk,), jnp.float32),
            pltpu.VMEM((chunk,), jnp.float32),
            pltpu.VMEM((chunk,), jnp.float32),
            pltpu.VMEM((chunk,), jnp.float32),
            pltpu.VMEM((8, _L), jnp.float32),
            pltpu.SemaphoreType.DMA,
            pltpu.SemaphoreType.DMA,
        ],
        interpret=interpret,
    )


_ROWS = 1024
_COLS = 8192
_SC_ROWS = 384
_SC_N = _SC_ROWS * _COLS
_TC_BR = 64
_TC_BLOCKS = (_ROWS - _SC_ROWS) // _TC_BR


def _tc_body(p_ref, t_ref, m_ref, s_ref, b_ref, a_ref, bgt_ref, agt_ref, cnt_ref):
    i = pl.program_id(0)

    @pl.when(i == 0)
    def _():
        m_ref[...] = jnp.full_like(m_ref, -1e30)
        s_ref[...] = jnp.zeros_like(s_ref)
        b_ref[...] = jnp.zeros_like(b_ref)
        a_ref[...] = jnp.zeros_like(a_ref)
        bgt_ref[...] = jnp.zeros_like(bgt_ref)
        agt_ref[...] = jnp.zeros_like(agt_ref)
        cnt_ref[...] = jnp.zeros_like(cnt_ref)

    p3 = p_ref[...].reshape(_TC_BR // 8, 8, _COLS)
    t3 = t_ref[...].reshape(_TC_BR // 8, 8, _COLS)
    z3 = jnp.zeros_like(t3)
    tp = t3 * p3
    gt = t3 > p3
    sg = 1.0 / (1.0 + jnp.exp(-p3))
    g = jnp.abs(sg - t3)
    b_ref[...] += jnp.sum(t3, axis=0)
    a_ref[...] += jnp.sum(tp, axis=0)
    bgt_ref[...] += jnp.sum(jnp.where(gt, t3, z3), axis=0)
    agt_ref[...] += jnp.sum(jnp.where(gt, tp, z3), axis=0)
    cnt_ref[...] += jnp.sum(jnp.where(g <= _EDGE10, jnp.ones_like(t3), z3), axis=0)
    M = m_ref[...]
    Mn = jnp.maximum(M, jnp.max(p3, axis=0))
    s_ref[...] = s_ref[...] * jnp.exp(M - Mn) + jnp.sum(jnp.exp(p3 - Mn[None]), axis=0)
    m_ref[...] = Mn


def _tc_partials(pred2d, target2d):
    acc = jax.ShapeDtypeStruct((8, _COLS), jnp.float32)
    in_spec = pl.BlockSpec((_TC_BR, _COLS), lambda i: (_SC_ROWS // _TC_BR + i, 0))
    out_spec = pl.BlockSpec((8, _COLS), lambda i: (0, 0))
    return pl.pallas_call(
        _tc_body,
        grid=(_TC_BLOCKS,),
        in_specs=[in_spec, in_spec],
        out_specs=[out_spec] * 7,
        out_shape=[acc] * 7,
    )(pred2d, target2d)


def _combine(part_sets, n_total):
    Mg = jnp.max(jnp.stack([jnp.max(ps[0]) for ps in part_sets]))
    S_tot = sum(jnp.sum(ps[1] * jnp.exp(ps[0] - Mg)) for ps in part_sets)
    logZ = Mg + jnp.log(S_tot)
    B = sum(jnp.sum(ps[2]) for ps in part_sets)
    A = sum(jnp.sum(ps[3]) for ps in part_sets)
    Bgt = sum(jnp.sum(ps[4]) for ps in part_sets)
    Agt = sum(jnp.sum(ps[5]) for ps in part_sets)
    cnt = sum(jnp.sum(ps[6]) for ps in part_sets)
    acc_sum = (1.0 - _MOMENTUM) * cnt
    total_neg = jnp.float32(n_total) - acc_sum
    total_pos = jnp.maximum(acc_sum, 1.0)
    ratio = total_neg / total_pos
    r1 = ratio - 1.0
    return -(A + r1 * Agt) + logZ * (B + r1 * Bgt)


@functools.lru_cache(maxsize=None)
def _sc_partials():
    return _make_sc_partials(_SC_N, 16384)


def kernel(pred, target):
    sc = _sc_partials()(pred, target)
    sc_set = tuple(sc[:, j, :] for j in range(7))
    tc_set = _tc_partials(pred.reshape(_ROWS, _COLS), target.reshape(_ROWS, _COLS))
    return _combine([sc_set, tuple(tc_set)], _N)


# speedup vs baseline: 1.9828x; 1.9828x over previous
import functools

import jax
import jax.numpy as jnp
import numpy as np
from jax import lax
from jax.experimental import pallas as pl
from jax.experimental.pallas import tpu as pltpu
from jax.experimental.pallas import tpu_sc as plsc

_N = 8388608
_NC = 2
_NS = 16
_NW = _NC * _NS
_L = 16

_EDGE10 = float(np.float32(1.0) + np.float32(1e-6))
_MOMENTUM = 0.5


def _make_sc_partials(n_total, chunk, interpret=False):
    per_w = n_total // _NW
    chunks = per_w // chunk
    vecs = chunk // _L
    assert per_w * _NW == n_total and chunks * chunk == per_w and vecs * _L == chunk

    assert chunks % 2 == 0

    def body(pred_hbm, target_hbm, out_hbm, pa, ta, pb, tb, acc_v, sem_a, sem_b):
        wid = lax.axis_index("s") * _NC + lax.axis_index("c")
        base = wid * per_w

        zeros = jnp.zeros((_L,), jnp.float32)
        ones = jnp.ones((_L,), jnp.float32)
        neg_big = jnp.full((_L,), -1e30, jnp.float32)

        def start(bp, bt, sem, c):
            s0 = base + c * chunk
            pltpu.async_copy(pred_hbm.at[pl.ds(s0, chunk)], bp, sem)
            pltpu.async_copy(target_hbm.at[pl.ds(s0, chunk)], bt, sem)

        def wait(bp, bt, sem, c):
            s0 = base + c * chunk
            pltpu.make_async_copy(pred_hbm.at[pl.ds(s0, chunk)], bp, sem).wait()
            pltpu.make_async_copy(target_hbm.at[pl.ds(s0, chunk)], bt, sem).wait()

        def compute(pbuf, tbuf, carry):
            M, S, B, A, Bgt, Agt, cnt = carry

            def pass1(i, c1):
                cm, B, A, Bgt, Agt, cnt = c1
                v = pbuf[pl.ds(i * _L, _L)]
                t = tbuf[pl.ds(i * _L, _L)]
                cm = jnp.maximum(cm, v)
                tp = t * v
                B = B + t
                A = A + tp
                gt = t > v
                Bgt = Bgt + jnp.where(gt, t, zeros)
                Agt = Agt + jnp.where(gt, tp, zeros)
                sg = 1.0 / (1.0 + jnp.exp(-v))
                g = jnp.abs(sg - t)
                cnt = cnt + jnp.where(g <= _EDGE10, ones, zeros)
                return (cm, B, A, Bgt, Agt, cnt)

            cm, B, A, Bgt, Agt, cnt = lax.fori_loop(
                0, vecs, pass1, (neg_big, B, A, Bgt, Agt, cnt), unroll=4)

            Mn = jnp.maximum(M, cm)
            S = S * jnp.exp(M - Mn)

            def pass2(i, s):
                v = pbuf[pl.ds(i * _L, _L)]
                return s + jnp.exp(v - Mn)

            S = lax.fori_loop(0, vecs, pass2, S, unroll=4)
            return (Mn, S, B, A, Bgt, Agt, cnt)

        start(pa, ta, sem_a, 0)

        def pair_body(i, carry):
            c0 = 2 * i
            start(pb, tb, sem_b, c0 + 1)
            wait(pa, ta, sem_a, c0)
            carry = compute(pa, ta, carry)

            @pl.when(c0 + 2 < chunks)
            def _():
                start(pa, ta, sem_a, c0 + 2)

            wait(pb, tb, sem_b, c0 + 1)
            carry = compute(pb, tb, carry)
            return carry

        init = (neg_big, zeros, zeros, zeros, zeros, zeros, zeros)
        M, S, B, A, Bgt, Agt, cnt = lax.fori_loop(0, chunks // 2, pair_body, init)

        acc_v[0] = M
        acc_v[1] = S
        acc_v[2] = B
        acc_v[3] = A
        acc_v[4] = Bgt
        acc_v[5] = Agt
        acc_v[6] = cnt
        acc_v[7] = zeros
        pltpu.sync_copy(acc_v, out_hbm.at[wid])

    return pl.kernel(
        body,
        out_type=jax.ShapeDtypeStruct((_NW, 8, _L), jnp.float32),
        mesh=plsc.VectorSubcoreMesh(
            core_axis_name="c", subcore_axis_name="s",
            num_cores=_NC, num_subcores=_NS),
        scratch_types=[
            pltpu.VMEM((chunk,), jnp.float32),
            pltpu.VMEM((chunk,), jnp.float32),
            pltpu.VMEM((chunk,), jnp.float32),
            pltpu.VMEM((chunk,), jnp.float32),
            pltpu.VMEM((8, _L), jnp.float32),
            pltpu.SemaphoreType.DMA,
            pltpu.SemaphoreType.DMA,
        ],
        interpret=interpret,
    )


_SC_N = 3145728
_TC_BN = 524288
_TC_BLOCKS = (_N - _SC_N) // _TC_BN


def _tc_body(p_ref, t_ref, m_ref, s_ref, b_ref, a_ref, bgt_ref, agt_ref, cnt_ref):
    i = pl.program_id(0)

    @pl.when(i == 0)
    def _():
        m_ref[...] = jnp.full_like(m_ref, -1e30)
        s_ref[...] = jnp.zeros_like(s_ref)
        b_ref[...] = jnp.zeros_like(b_ref)
        a_ref[...] = jnp.zeros_like(a_ref)
        bgt_ref[...] = jnp.zeros_like(bgt_ref)
        agt_ref[...] = jnp.zeros_like(agt_ref)
        cnt_ref[...] = jnp.zeros_like(cnt_ref)

    p3 = p_ref[...].reshape(_TC_BN // 1024, 8, 128)
    t3 = t_ref[...].reshape(_TC_BN // 1024, 8, 128)
    z3 = jnp.zeros_like(t3)
    tp = t3 * p3
    gt = t3 > p3
    sg = 1.0 / (1.0 + jnp.exp(-p3))
    g = jnp.abs(sg - t3)
    b_ref[...] += jnp.sum(t3, axis=0)
    a_ref[...] += jnp.sum(tp, axis=0)
    bgt_ref[...] += jnp.sum(jnp.where(gt, t3, z3), axis=0)
    agt_ref[...] += jnp.sum(jnp.where(gt, tp, z3), axis=0)
    cnt_ref[...] += jnp.sum(jnp.where(g <= _EDGE10, jnp.ones_like(t3), z3), axis=0)
    M = m_ref[...]
    Mn = jnp.maximum(M, jnp.max(p3, axis=0))
    s_ref[...] = s_ref[...] * jnp.exp(M - Mn) + jnp.sum(jnp.exp(p3 - Mn[None]), axis=0)
    m_ref[...] = Mn


def _tc_partials(pred, target):
    acc = jax.ShapeDtypeStruct((8, 128), jnp.float32)
    in_spec = pl.BlockSpec((_TC_BN,), lambda i: (_SC_N // _TC_BN + i,))
    out_spec = pl.BlockSpec((8, 128), lambda i: (0, 0))
    return pl.pallas_call(
        _tc_body,
        grid=(_TC_BLOCKS,),
        in_specs=[in_spec, in_spec],
        out_specs=[out_spec] * 7,
        out_shape=[acc] * 7,
    )(pred, target)


def _combine(part_sets, n_total):
    Mg = jnp.max(jnp.stack([jnp.max(ps[0]) for ps in part_sets]))
    S_tot = sum(jnp.sum(ps[1] * jnp.exp(ps[0] - Mg)) for ps in part_sets)
    logZ = Mg + jnp.log(S_tot)
    B = sum(jnp.sum(ps[2]) for ps in part_sets)
    A = sum(jnp.sum(ps[3]) for ps in part_sets)
    Bgt = sum(jnp.sum(ps[4]) for ps in part_sets)
    Agt = sum(jnp.sum(ps[5]) for ps in part_sets)
    cnt = sum(jnp.sum(ps[6]) for ps in part_sets)
    acc_sum = (1.0 - _MOMENTUM) * cnt
    total_neg = jnp.float32(n_total) - acc_sum
    total_pos = jnp.maximum(acc_sum, 1.0)
    ratio = total_neg / total_pos
    r1 = ratio - 1.0
    return -(A + r1 * Agt) + logZ * (B + r1 * Bgt)


@functools.lru_cache(maxsize=None)
def _sc_partials():
    return _make_sc_partials(_SC_N, 16384)


def kernel(pred, target):
    sc = _sc_partials()(pred, target)
    sc_set = tuple(sc[:, j, :] for j in range(7))
    tc_set = _tc_partials(pred, target)
    return _combine([sc_set, tuple(tc_set)], _N)


# speedup vs baseline: 2.7157x; 1.3696x over previous
import functools

import jax
import jax.numpy as jnp
import numpy as np
from jax import lax
from jax.experimental import pallas as pl
from jax.experimental.pallas import tpu as pltpu
from jax.experimental.pallas import tpu_sc as plsc

_N = 8388608
_NC = 2
_NS = 16
_NW = _NC * _NS
_L = 16

_EDGE10 = float(np.float32(1.0) + np.float32(1e-6))
_MOMENTUM = 0.5


def _make_sc_partials(n_total, chunk, interpret=False):
    per_w = n_total // _NW
    chunks = per_w // chunk
    vecs = chunk // _L
    assert per_w * _NW == n_total and chunks * chunk == per_w and vecs * _L == chunk

    assert chunks % 2 == 0

    def body(pred_hbm, target_hbm, out_hbm, pa, ta, pb, tb, acc_v, sem_a, sem_b):
        wid = lax.axis_index("s") * _NC + lax.axis_index("c")
        base = wid * per_w

        zeros = jnp.zeros((_L,), jnp.float32)
        ones = jnp.ones((_L,), jnp.float32)
        neg_big = jnp.full((_L,), -1e30, jnp.float32)

        def start(bp, bt, sem, c):
            s0 = base + c * chunk
            pltpu.async_copy(pred_hbm.at[pl.ds(s0, chunk)], bp, sem)
            pltpu.async_copy(target_hbm.at[pl.ds(s0, chunk)], bt, sem)

        def wait(bp, bt, sem, c):
            s0 = base + c * chunk
            pltpu.make_async_copy(pred_hbm.at[pl.ds(s0, chunk)], bp, sem).wait()
            pltpu.make_async_copy(target_hbm.at[pl.ds(s0, chunk)], bt, sem).wait()

        def compute(pbuf, tbuf, carry):
            M, S, B, A, Bgt, Agt, cnt = carry

            def pass1(i, c1):
                cm, B, A, Bgt, Agt, cnt = c1
                v = pbuf[pl.ds(i * _L, _L)]
                t = tbuf[pl.ds(i * _L, _L)]
                cm = jnp.maximum(cm, v)
                tp = t * v
                B = B + t
                A = A + tp
                gt = t > v
                Bgt = Bgt + jnp.where(gt, t, zeros)
                Agt = Agt + jnp.where(gt, tp, zeros)
                sg = 1.0 / (1.0 + jnp.exp(-v))
                g = jnp.abs(sg - t)
                cnt = cnt + jnp.where(g <= _EDGE10, ones, zeros)
                return (cm, B, A, Bgt, Agt, cnt)

            cm, B, A, Bgt, Agt, cnt = lax.fori_loop(
                0, vecs, pass1, (neg_big, B, A, Bgt, Agt, cnt), unroll=4)

            Mn = jnp.maximum(M, cm)
            S = S * jnp.exp(M - Mn)

            def pass2(i, s):
                v = pbuf[pl.ds(i * _L, _L)]
                return s + jnp.exp(v - Mn)

            S = lax.fori_loop(0, vecs, pass2, S, unroll=4)
            return (Mn, S, B, A, Bgt, Agt, cnt)

        start(pa, ta, sem_a, 0)

        def pair_body(i, carry):
            c0 = 2 * i
            start(pb, tb, sem_b, c0 + 1)
            wait(pa, ta, sem_a, c0)
            carry = compute(pa, ta, carry)

            @pl.when(c0 + 2 < chunks)
            def _():
                start(pa, ta, sem_a, c0 + 2)

            wait(pb, tb, sem_b, c0 + 1)
            carry = compute(pb, tb, carry)
            return carry

        init = (neg_big, zeros, zeros, zeros, zeros, zeros, zeros)
        M, S, B, A, Bgt, Agt, cnt = lax.fori_loop(0, chunks // 2, pair_body, init)

        acc_v[0] = M
        acc_v[1] = S
        acc_v[2] = B
        acc_v[3] = A
        acc_v[4] = Bgt
        acc_v[5] = Agt
        acc_v[6] = cnt
        acc_v[7] = zeros
        pltpu.sync_copy(acc_v, out_hbm.at[wid])

    return pl.kernel(
        body,
        out_type=jax.ShapeDtypeStruct((_NW, 8, _L), jnp.float32),
        mesh=plsc.VectorSubcoreMesh(
            core_axis_name="c", subcore_axis_name="s",
            num_cores=_NC, num_subcores=_NS),
        scratch_types=[
            pltpu.VMEM((chunk,), jnp.float32),
            pltpu.VMEM((chunk,), jnp.float32),
            pltpu.VMEM((chunk,), jnp.float32),
            pltpu.VMEM((chunk,), jnp.float32),
            pltpu.VMEM((8, _L), jnp.float32),
            pltpu.SemaphoreType.DMA,
            pltpu.SemaphoreType.DMA,
        ],
        interpret=interpret,
    )


_SC_N = 2621440
_TC_BN = 524288
_TC_BLOCKS = (_N - _SC_N) // _TC_BN


def _tc_body(p_ref, t_ref, m_ref, s_ref, b_ref, a_ref, bgt_ref, agt_ref, cnt_ref):
    i = pl.program_id(0)

    @pl.when(i == 0)
    def _():
        m_ref[...] = jnp.full_like(m_ref, -1e30)
        s_ref[...] = jnp.zeros_like(s_ref)
        b_ref[...] = jnp.zeros_like(b_ref)
        a_ref[...] = jnp.zeros_like(a_ref)
        bgt_ref[...] = jnp.zeros_like(bgt_ref)
        agt_ref[...] = jnp.zeros_like(agt_ref)
        cnt_ref[...] = jnp.zeros_like(cnt_ref)

    p3 = p_ref[...].reshape(_TC_BN // 1024, 8, 128)
    t3 = t_ref[...].reshape(_TC_BN // 1024, 8, 128)
    z3 = jnp.zeros_like(t3)
    tp = t3 * p3
    gt = t3 > p3
    sg = 1.0 / (1.0 + jnp.exp(-p3))
    g = jnp.abs(sg - t3)
    b_ref[...] += jnp.sum(t3, axis=0)
    a_ref[...] += jnp.sum(tp, axis=0)
    bgt_ref[...] += jnp.sum(jnp.where(gt, t3, z3), axis=0)
    agt_ref[...] += jnp.sum(jnp.where(gt, tp, z3), axis=0)
    cnt_ref[...] += jnp.sum(jnp.where(g <= _EDGE10, jnp.ones_like(t3), z3), axis=0)
    M = m_ref[...]
    Mn = jnp.maximum(M, jnp.max(p3, axis=0))
    s_ref[...] = s_ref[...] * jnp.exp(M - Mn) + jnp.sum(jnp.exp(p3 - Mn[None]), axis=0)
    m_ref[...] = Mn


def _tc_partials(pred, target):
    acc = jax.ShapeDtypeStruct((8, 128), jnp.float32)
    in_spec = pl.BlockSpec((_TC_BN,), lambda i: (_SC_N // _TC_BN + i,))
    out_spec = pl.BlockSpec((8, 128), lambda i: (0, 0))
    return pl.pallas_call(
        _tc_body,
        grid=(_TC_BLOCKS,),
        in_specs=[in_spec, in_spec],
        out_specs=[out_spec] * 7,
        out_shape=[acc] * 7,
    )(pred, target)


def _combine_body(sc_ref, m_ref, s_ref, b_ref, a_ref, bgt_ref, agt_ref,
                  cnt_ref, out_ref):
    sc = sc_ref[...]
    M_tc = m_ref[...]
    Mg = jnp.maximum(jnp.max(sc[:, 0, :]), jnp.max(M_tc))
    S_tot = (jnp.sum(sc[:, 1, :] * jnp.exp(sc[:, 0, :] - Mg))
             + jnp.sum(s_ref[...] * jnp.exp(M_tc - Mg)))
    logZ = Mg + jnp.log(S_tot)
    B = jnp.sum(sc[:, 2, :]) + jnp.sum(b_ref[...])
    A = jnp.sum(sc[:, 3, :]) + jnp.sum(a_ref[...])
    Bgt = jnp.sum(sc[:, 4, :]) + jnp.sum(bgt_ref[...])
    Agt = jnp.sum(sc[:, 5, :]) + jnp.sum(agt_ref[...])
    cnt = jnp.sum(sc[:, 6, :]) + jnp.sum(cnt_ref[...])
    acc_sum = (1.0 - _MOMENTUM) * cnt
    total_neg = jnp.float32(_N) - acc_sum
    total_pos = jnp.maximum(acc_sum, 1.0)
    ratio = total_neg / total_pos
    r1 = ratio - 1.0
    loss = -(A + r1 * Agt) + logZ * (B + r1 * Bgt)
    out_ref[...] = jnp.full((8, 128), loss, jnp.float32)


def _combine_call(sc_parts, tc_parts):
    return pl.pallas_call(
        _combine_body,
        out_shape=jax.ShapeDtypeStruct((8, 128), jnp.float32),
    )(sc_parts, *tc_parts)


@functools.lru_cache(maxsize=None)
def _sc_partials():
    return _make_sc_partials(_SC_N, 8192)


def kernel(pred, target):
    sc = _sc_partials()(pred, target)
    tc = _tc_partials(pred, target)
    return _combine_call(sc, tc)[0, 0]


# speedup vs baseline: 2.7687x; 1.0195x over previous
import functools

import jax
import jax.numpy as jnp
import numpy as np
from jax import lax
from jax.experimental import pallas as pl
from jax.experimental.pallas import tpu as pltpu
from jax.experimental.pallas import tpu_sc as plsc

_N = 8388608
_NC = 2
_NS = 16
_NW = _NC * _NS
_L = 16

_EDGE10 = float(np.float32(1.0) + np.float32(1e-6))
_MOMENTUM = 0.5


def _make_sc_partials(n_total, chunk):
    per_w = n_total // _NW
    chunks = per_w // chunk
    vecs = chunk // _L
    assert per_w * _NW == n_total and chunks * chunk == per_w and vecs * _L == chunk

    assert chunks % 2 == 0

    def body(pred_hbm, target_hbm, out_hbm, pa, ta, pb, tb, acc_v, sem_a, sem_b):
        wid = lax.axis_index("s") * _NC + lax.axis_index("c")
        base = wid * per_w

        zeros = jnp.zeros((_L,), jnp.float32)
        ones = jnp.ones((_L,), jnp.float32)
        neg_big = jnp.full((_L,), -1e30, jnp.float32)

        def start(bp, bt, sem, c):
            s0 = base + c * chunk
            pltpu.async_copy(pred_hbm.at[pl.ds(s0, chunk)], bp, sem)
            pltpu.async_copy(target_hbm.at[pl.ds(s0, chunk)], bt, sem)

        def wait(bp, bt, sem, c):
            s0 = base + c * chunk
            pltpu.make_async_copy(pred_hbm.at[pl.ds(s0, chunk)], bp, sem).wait()
            pltpu.make_async_copy(target_hbm.at[pl.ds(s0, chunk)], bt, sem).wait()

        def compute(pbuf, tbuf, carry):
            M, S, B, A, Bgt, Agt, cnt = carry

            def pass1(i, c1):
                cm, B, A, Bgt, Agt, cnt = c1
                v = pbuf[pl.ds(i * _L, _L)]
                t = tbuf[pl.ds(i * _L, _L)]
                cm = jnp.maximum(cm, v)
                tp = t * v
                B = B + t
                A = A + tp
                gt = t > v
                Bgt = Bgt + jnp.where(gt, t, zeros)
                Agt = Agt + jnp.where(gt, tp, zeros)
                sg = 1.0 / (1.0 + jnp.exp(-v))
                g = jnp.abs(sg - t)
                cnt = cnt + jnp.where(g <= _EDGE10, ones, zeros)
                return (cm, B, A, Bgt, Agt, cnt)

            cm, B, A, Bgt, Agt, cnt = lax.fori_loop(
                0, vecs, pass1, (neg_big, B, A, Bgt, Agt, cnt), unroll=4)

            Mn = jnp.maximum(M, cm)
            S = S * jnp.exp(M - Mn)

            def pass2(i, s):
                v = pbuf[pl.ds(i * _L, _L)]
                return s + jnp.exp(v - Mn)

            S = lax.fori_loop(0, vecs, pass2, S, unroll=4)
            return (Mn, S, B, A, Bgt, Agt, cnt)

        start(pa, ta, sem_a, 0)

        def pair_body(i, carry):
            c0 = 2 * i
            start(pb, tb, sem_b, c0 + 1)
            wait(pa, ta, sem_a, c0)
            carry = compute(pa, ta, carry)

            @pl.when(c0 + 2 < chunks)
            def _():
                start(pa, ta, sem_a, c0 + 2)

            wait(pb, tb, sem_b, c0 + 1)
            carry = compute(pb, tb, carry)
            return carry

        init = (neg_big, zeros, zeros, zeros, zeros, zeros, zeros)
        M, S, B, A, Bgt, Agt, cnt = lax.fori_loop(0, chunks // 2, pair_body, init)

        acc_v[0] = M
        acc_v[1] = S
        acc_v[2] = B
        acc_v[3] = A
        acc_v[4] = Bgt
        acc_v[5] = Agt
        acc_v[6] = cnt
        acc_v[7] = zeros
        pltpu.sync_copy(acc_v, out_hbm.at[wid])

    return pl.kernel(
        body,
        out_type=jax.ShapeDtypeStruct((_NW, 8, _L), jnp.float32),
        mesh=plsc.VectorSubcoreMesh(
            core_axis_name="c", subcore_axis_name="s",
            num_cores=_NC, num_subcores=_NS),
        scratch_types=[
            pltpu.VMEM((chunk,), jnp.float32),
            pltpu.VMEM((chunk,), jnp.float32),
            pltpu.VMEM((chunk,), jnp.float32),
            pltpu.VMEM((chunk,), jnp.float32),
            pltpu.VMEM((8, _L), jnp.float32),
            pltpu.SemaphoreType.DMA,
            pltpu.SemaphoreType.DMA,
        ],
    )


_SC_N = 2621440
_TC_BN = 524288
_TC_BLOCKS = (_N - _SC_N) // _TC_BN


def _tc_body(p_ref, t_ref, out_ref,
             m_v, s_v, b_v, a_v, bgt_v, agt_v, cnt_v):
    i = pl.program_id(0)

    @pl.when(i == 0)
    def _():
        m_v[...] = jnp.full_like(m_v, -1e30)
        s_v[...] = jnp.zeros_like(s_v)
        b_v[...] = jnp.zeros_like(b_v)
        a_v[...] = jnp.zeros_like(a_v)
        bgt_v[...] = jnp.zeros_like(bgt_v)
        agt_v[...] = jnp.zeros_like(agt_v)
        cnt_v[...] = jnp.zeros_like(cnt_v)

    p3 = p_ref[...].reshape(_TC_BN // 1024, 8, 128)
    t3 = t_ref[...].reshape(_TC_BN // 1024, 8, 128)
    z3 = jnp.zeros_like(t3)
    tp = t3 * p3
    gt = t3 > p3
    sg = 1.0 / (1.0 + jnp.exp(-p3))
    g = jnp.abs(sg - t3)
    b_v[...] += jnp.sum(t3, axis=0)
    a_v[...] += jnp.sum(tp, axis=0)
    bgt_v[...] += jnp.sum(jnp.where(gt, t3, z3), axis=0)
    agt_v[...] += jnp.sum(jnp.where(gt, tp, z3), axis=0)
    cnt_v[...] += jnp.sum(jnp.where(g <= _EDGE10, jnp.ones_like(t3), z3), axis=0)
    M = m_v[...]
    Mn = jnp.maximum(M, jnp.max(p3, axis=0))
    s_v[...] = s_v[...] * jnp.exp(M - Mn) + jnp.sum(jnp.exp(p3 - Mn[None]), axis=0)
    m_v[...] = Mn

    @pl.when(i == _TC_BLOCKS - 1)
    def _():
        out_ref[0] = m_v[...]
        out_ref[1] = s_v[...]
        out_ref[2] = b_v[...]
        out_ref[3] = a_v[...]
        out_ref[4] = bgt_v[...]
        out_ref[5] = agt_v[...]
        out_ref[6] = cnt_v[...]


def _tc_partials(pred, target):
    in_spec = pl.BlockSpec((_TC_BN,), lambda i: (_SC_N // _TC_BN + i,))
    out_spec = pl.BlockSpec((7, 8, 128), lambda i: (0, 0, 0))
    return pl.pallas_call(
        _tc_body,
        grid=(_TC_BLOCKS,),
        in_specs=[in_spec, in_spec],
        out_specs=out_spec,
        out_shape=jax.ShapeDtypeStruct((7, 8, 128), jnp.float32),
        scratch_shapes=[pltpu.VMEM((8, 128), jnp.float32)] * 7,
    )(pred, target)


def _combine_body(sc_ref, tc_ref, out_ref):
    sc = sc_ref[...]
    M_tc = tc_ref[0]
    Mg = jnp.maximum(jnp.max(sc[:, 0, :]), jnp.max(M_tc))
    S_tot = (jnp.sum(sc[:, 1, :] * jnp.exp(sc[:, 0, :] - Mg))
             + jnp.sum(tc_ref[1] * jnp.exp(M_tc - Mg)))
    logZ = Mg + jnp.log(S_tot)
    B = jnp.sum(sc[:, 2, :]) + jnp.sum(tc_ref[2])
    A = jnp.sum(sc[:, 3, :]) + jnp.sum(tc_ref[3])
    Bgt = jnp.sum(sc[:, 4, :]) + jnp.sum(tc_ref[4])
    Agt = jnp.sum(sc[:, 5, :]) + jnp.sum(tc_ref[5])
    cnt = jnp.sum(sc[:, 6, :]) + jnp.sum(tc_ref[6])
    acc_sum = (1.0 - _MOMENTUM) * cnt
    total_neg = jnp.float32(_N) - acc_sum
    total_pos = jnp.maximum(acc_sum, 1.0)
    ratio = total_neg / total_pos
    r1 = ratio - 1.0
    loss = -(A + r1 * Agt) + logZ * (B + r1 * Bgt)
    out_ref[...] = loss


def _combine_call(sc_parts, tc_parts):
    return pl.pallas_call(
        _combine_body,
        out_shape=jax.ShapeDtypeStruct((), jnp.float32),
        out_specs=pl.BlockSpec(memory_space=pltpu.SMEM),
    )(sc_parts, tc_parts)


@functools.lru_cache(maxsize=None)
def _sc_partials():
    return _make_sc_partials(_SC_N, 8192)


def kernel(pred, target):
    sc = _sc_partials()(pred, target)
    tc = _tc_partials(pred, target)
    return _combine_call(sc, tc)
